# baseline (speedup 1.0000x reference)
import functools

import jax
import jax.numpy as jnp
from jax import lax
from jax.experimental import pallas as pl
from jax.experimental.pallas import tpu as pltpu
from jax.experimental.pallas import tpu_sc as plsc

NC = 2
NS = 16
NW = NC * NS

CHUNK = 128
NBUF = 4
LEAD = 3


@functools.partial(jax.jit, static_argnums=(2, 3, 4))
def _gather(x_flat, weight, n_tok, seq, d):
    b_per_w = n_tok * seq // NW
    nblock = b_per_w // CHUNK
    ngroup = nblock // NBUF
    blk_per_s = n_tok // CHUNK
    dt = d // 8
    mesh = plsc.VectorSubcoreMesh(core_axis_name="c", subcore_axis_name="s")

    @functools.partial(
        pl.kernel,
        out_type=jax.ShapeDtypeStruct((seq * dt * blk_per_s, 8 * CHUNK), jnp.float32),
        mesh=mesh,
        scratch_types=(
            [pltpu.VMEM((b_per_w,), jnp.int32)]
            + [pltpu.VMEM((CHUNK, d), jnp.float32) for _ in range(NBUF)]
            + [pltpu.VMEM((d * CHUNK,), jnp.float32) for _ in range(NBUF)]
            + [pltpu.SemaphoreType.DMA for _ in range(2 * NBUF)]
        ),
        compiler_params=pltpu.CompilerParams(
            use_tc_tiling_on_sc=False, needs_layout_passes=False
        ),
    )
    def k(x_hbm, w_hbm, out_hbm, idx_v, *scratch):
        gbuf = scratch[:NBUF]
        tbuf = scratch[NBUF:2 * NBUF]
        gsem = scratch[2 * NBUF:3 * NBUF]
        ssem = scratch[3 * NBUF:]
        wid = lax.axis_index("s") * NC + lax.axis_index("c")
        base_blk = wid * nblock
        pltpu.sync_copy(x_hbm.at[pl.ds(base_blk * CHUNK, b_per_w)], idx_v)

        def fire(b, i):
            pltpu.async_copy(
                w_hbm.at[idx_v.at[pl.ds(i * CHUNK, CHUNK)]], gbuf[b], gsem[b]
            )

        def wait_gather(b):
            pltpu.make_async_copy(
                w_hbm.at[pl.ds(0, CHUNK)], gbuf[b], gsem[b]
            ).wait()

        def wait_stores(b):
            for jt in range(dt):
                pltpu.make_async_copy(
                    tbuf[b].at[pl.ds(jt * 8 * CHUNK, 8 * CHUNK)],
                    out_hbm.at[0],
                    ssem[b],
                ).wait()

        for b in range(LEAD):
            fire(b, b)

        row16 = lax.iota(jnp.int32, 16)

        @pl.loop(0, ngroup)
        def _(g):
            for b in range(NBUF):
                i = g * NBUF + b
                gblk = base_blk + i
                s = gblk // blk_per_s
                bt = gblk % blk_per_s
                wait_gather(b)

                @pl.when(i >= NBUF)
                def _():
                    wait_stores(b)

                @pl.loop(0, CHUNK)
                def _(bq):
                    for j in range(d // 16):
                        v = gbuf[b][bq, pl.ds(16 * j, 16)]
                        plsc.store_scatter(
                            tbuf[b], [(row16 + (16 * j)) * CHUNK + bq], v
                        )

                for jt in range(dt):
                    pltpu.async_copy(
                        tbuf[b].at[pl.ds(jt * 8 * CHUNK, 8 * CHUNK)],
                        out_hbm.at[(s * dt + jt) * blk_per_s + bt],
                        ssem[b],
                    )

                nxt = i + LEAD

                @pl.when(nxt < nblock)
                def _():
                    fire((b + LEAD) % NBUF, nxt)

        for b in range(NBUF):
            wait_stores(b)

    return k(x_flat, weight)


def kernel(x, weight):
    n, seq = x.shape
    d = weight.shape[1]
    x_flat = x.T.reshape(n * seq).astype(jnp.int32)
    out = _gather(x_flat, weight, n, seq, d)
    return (
        out.reshape(seq, d // 8, n // CHUNK, 8, CHUNK)
        .transpose(2, 4, 0, 1, 3)
        .reshape(n, seq, d)
    )


# speedup vs baseline: 1.2155x; 1.2155x over previous
import functools

import jax
import jax.numpy as jnp
from jax import lax
from jax.experimental import pallas as pl
from jax.experimental.pallas import tpu as pltpu
from jax.experimental.pallas import tpu_sc as plsc

NC = 2
NS = 16
NW = NC * NS

CHUNK = 128
NBUF = 4
LEAD = 3


@functools.partial(jax.jit, static_argnums=(2, 3, 4))
def _gather(x_flat, weight, n_tok, seq, d):
    b_per_w = n_tok * seq // NW
    nblock = b_per_w // CHUNK
    ngroup = nblock // NBUF
    blk_per_s = n_tok // CHUNK
    dt = d // 8
    mesh = plsc.VectorSubcoreMesh(core_axis_name="c", subcore_axis_name="s")

    @functools.partial(
        pl.kernel,
        out_type=jax.ShapeDtypeStruct((seq * dt * blk_per_s, 8 * CHUNK), jnp.float32),
        mesh=mesh,
        scratch_types=(
            [pltpu.VMEM((b_per_w,), jnp.int32)]
            + [pltpu.VMEM((CHUNK, d), jnp.float32) for _ in range(NBUF)]
            + [pltpu.VMEM((d * CHUNK,), jnp.float32) for _ in range(NBUF)]
            + [pltpu.SemaphoreType.DMA for _ in range(2 * NBUF)]
        ),
        compiler_params=pltpu.CompilerParams(
            use_tc_tiling_on_sc=False, needs_layout_passes=False
        ),
    )
    def k(x_hbm, w_hbm, out_hbm, idx_v, *scratch):
        gbuf = scratch[:NBUF]
        tbuf = scratch[NBUF:2 * NBUF]
        gsem = scratch[2 * NBUF:3 * NBUF]
        ssem = scratch[3 * NBUF:]
        wid = lax.axis_index("s") * NC + lax.axis_index("c")
        base_blk = wid * nblock
        pltpu.sync_copy(x_hbm.at[pl.ds(base_blk * CHUNK, b_per_w)], idx_v)

        def fire(b, i):
            pltpu.async_copy(
                w_hbm.at[idx_v.at[pl.ds(i * CHUNK, CHUNK)]], gbuf[b], gsem[b]
            )

        def wait_gather(b):
            pltpu.make_async_copy(
                w_hbm.at[pl.ds(0, CHUNK)], gbuf[b], gsem[b]
            ).wait()

        def wait_stores(b):
            for jt in range(dt):
                pltpu.make_async_copy(
                    tbuf[b].at[pl.ds(jt * 8 * CHUNK, 8 * CHUNK)],
                    out_hbm.at[0],
                    ssem[b],
                ).wait()

        for b in range(LEAD):
            fire(b, b)

        row16 = lax.iota(jnp.int32, 16)

        @pl.loop(0, ngroup)
        def _(g):
            for b in range(NBUF):
                i = g * NBUF + b
                gblk = base_blk + i
                s = gblk // blk_per_s
                bt = gblk % blk_per_s
                wait_gather(b)

                @pl.when(i >= NBUF)
                def _():
                    wait_stores(b)

                @plsc.parallel_loop(0, CHUNK, unroll=16)
                def _(bq):
                    for j in range(d // 16):
                        v = gbuf[b][bq, pl.ds(16 * j, 16)]
                        plsc.store_scatter(
                            tbuf[b], [(row16 + (16 * j)) * CHUNK + bq], v
                        )

                for jt in range(dt):
                    pltpu.async_copy(
                        tbuf[b].at[pl.ds(jt * 8 * CHUNK, 8 * CHUNK)],
                        out_hbm.at[(s * dt + jt) * blk_per_s + bt],
                        ssem[b],
                    )

                nxt = i + LEAD

                @pl.when(nxt < nblock)
                def _():
                    fire((b + LEAD) % NBUF, nxt)

        for b in range(NBUF):
            wait_stores(b)

    return k(x_flat, weight)


def kernel(x, weight):
    n, seq = x.shape
    d = weight.shape[1]
    x_flat = x.T.reshape(n * seq).astype(jnp.int32)
    out = _gather(x_flat, weight, n, seq, d)
    return (
        out.reshape(seq, d // 8, n // CHUNK, 8, CHUNK)
        .transpose(2, 4, 0, 1, 3)
        .reshape(n, seq, d)
    )


# speedup vs baseline: 2.1247x; 1.7480x over previous
import functools

import jax
import jax.numpy as jnp
from jax import lax
from jax.experimental import pallas as pl
from jax.experimental.pallas import tpu as pltpu
from jax.experimental.pallas import tpu_sc as plsc

NC = 2
NS = 16
NW = NC * NS

CHUNK = 128
NBUF = 4
LEAD = 3


@functools.partial(jax.jit, static_argnums=(2, 3, 4))
def _gather(x_flat, weight, n_tok, seq, d):
    b_per_w = n_tok * seq // NW
    nblock = b_per_w // CHUNK
    ngroup = nblock // NBUF
    blk_per_s = n_tok // CHUNK
    dt = d // 8
    mesh = plsc.VectorSubcoreMesh(core_axis_name="c", subcore_axis_name="s")

    @functools.partial(
        pl.kernel,
        out_type=jax.ShapeDtypeStruct((seq * dt * blk_per_s, 8, CHUNK), jnp.float32),
        mesh=mesh,
        scratch_types=(
            [pltpu.VMEM((b_per_w,), jnp.int32)]
            + [pltpu.VMEM((CHUNK, d), jnp.float32) for _ in range(NBUF)]
            + [pltpu.VMEM((d, CHUNK + 1), jnp.float32) for _ in range(NBUF)]
            + [pltpu.SemaphoreType.DMA for _ in range(2 * NBUF)]
        ),
        compiler_params=pltpu.CompilerParams(
            use_tc_tiling_on_sc=False, needs_layout_passes=False
        ),
    )
    def k(x_hbm, w_hbm, out_hbm, idx_v, *scratch):
        gbuf = scratch[:NBUF]
        tbuf = scratch[NBUF:2 * NBUF]
        gsem = scratch[2 * NBUF:3 * NBUF]
        ssem = scratch[3 * NBUF:]
        wid = lax.axis_index("s") * NC + lax.axis_index("c")
        base_blk = wid * nblock
        pltpu.sync_copy(x_hbm.at[pl.ds(base_blk * CHUNK, b_per_w)], idx_v)

        def fire(b, i):
            pltpu.async_copy(
                w_hbm.at[idx_v.at[pl.ds(i * CHUNK, CHUNK)]], gbuf[b], gsem[b]
            )

        def wait_gather(b):
            pltpu.make_async_copy(
                w_hbm.at[pl.ds(0, CHUNK)], gbuf[b], gsem[b]
            ).wait()

        def wait_stores(b):
            for jt in range(dt):
                pltpu.make_async_copy(
                    tbuf[b].at[pl.ds(jt * 8, 8), pl.ds(0, CHUNK)],
                    out_hbm.at[0],
                    ssem[b],
                ).wait()

        for b in range(LEAD):
            fire(b, b)

        row16 = lax.iota(jnp.int32, 16)

        @pl.loop(0, ngroup)
        def _(g):
            for b in range(NBUF):
                i = g * NBUF + b
                gblk = base_blk + i
                s = gblk // blk_per_s
                bt = gblk % blk_per_s
                wait_gather(b)

                @pl.when(i >= NBUF)
                def _():
                    wait_stores(b)

                @plsc.parallel_loop(0, CHUNK, unroll=8)
                def _(bq):
                    colv = row16 * 0 + bq
                    for j in range(d // 16):
                        v = gbuf[b][bq, pl.ds(16 * j, 16)]
                        plsc.store_scatter(
                            tbuf[b], [row16 + (16 * j), colv], v
                        )

                for jt in range(dt):
                    pltpu.async_copy(
                        tbuf[b].at[pl.ds(jt * 8, 8), pl.ds(0, CHUNK)],
                        out_hbm.at[(s * dt + jt) * blk_per_s + bt],
                        ssem[b],
                    )

                nxt = i + LEAD

                @pl.when(nxt < nblock)
                def _():
                    fire((b + LEAD) % NBUF, nxt)

        for b in range(NBUF):
            wait_stores(b)

    return k(x_flat, weight)


def kernel(x, weight):
    n, seq = x.shape
    d = weight.shape[1]
    x_flat = x.T.reshape(n * seq).astype(jnp.int32)
    out = _gather(x_flat, weight, n, seq, d)
    return (
        out.reshape(seq, d // 8, n // CHUNK, 8, CHUNK)
        .transpose(2, 4, 0, 1, 3)
        .reshape(n, seq, d)
    )


# speedup vs baseline: 3.1577x; 1.4862x over previous
import functools

import jax
import jax.numpy as jnp
from jax import lax
from jax.experimental import pallas as pl
from jax.experimental.pallas import tpu as pltpu
from jax.experimental.pallas import tpu_sc as plsc

NC = 2
NS = 16
NW = NC * NS

CHUNK = 128
NBUF = 4
LEAD = 3
ABUF = 3


def _relayout(wt, wtail):
    d, v = wt.shape
    nfull = v // CHUNK
    mesh = plsc.VectorSubcoreMesh(core_axis_name="c", subcore_axis_name="s")

    @functools.partial(
        pl.kernel,
        out_type=jax.ShapeDtypeStruct((v // 2, 128), jnp.float32),
        mesh=mesh,
        scratch_types=(
            [pltpu.VMEM((d, 128), jnp.float32) for _ in range(ABUF)]
            + [pltpu.VMEM((64, 128), jnp.float32) for _ in range(ABUF)]
            + [pltpu.SemaphoreType.DMA for _ in range(2 * ABUF + 1)]
        ),
        compiler_params=pltpu.CompilerParams(
            use_tc_tiling_on_sc=True, needs_layout_passes=False
        ),
    )
    def ka(wt_hbm, wtail_hbm, out_hbm, *scratch):
        gbuf = scratch[:ABUF]
        tbuf = scratch[ABUF:2 * ABUF]
        gsem = scratch[2 * ABUF:3 * ABUF]
        ssem = scratch[3 * ABUF:4 * ABUF]
        tsem = scratch[4 * ABUF]
        wid = lax.axis_index("s") * NC + lax.axis_index("c")
        nblk = (nfull - wid + NW - 1) // NW
        row16 = lax.iota(jnp.int32, 16)

        @pl.when(wid == NW - 1)
        def _():
            pltpu.async_copy(wtail_hbm, gbuf[0].at[pl.ds(0, 32)], tsem)
            pltpu.make_async_copy(
                wtail_hbm, gbuf[0].at[pl.ds(0, 32)], tsem
            ).wait()
            pltpu.async_copy(
                gbuf[0].at[pl.ds(0, 32)],
                out_hbm.at[pl.ds(nfull * 64, 32)],
                tsem,
            )
            pltpu.make_async_copy(
                gbuf[0].at[pl.ds(0, 32)],
                out_hbm.at[pl.ds(nfull * 64, 32)],
                tsem,
            ).wait()

        def fire(b, t):
            pltpu.async_copy(
                wt_hbm.at[pl.ds(0, d), pl.ds((wid + t * NW) * CHUNK, CHUNK)],
                gbuf[b], gsem[b],
            )

        def wait_gather(b):
            pltpu.make_async_copy(
                wt_hbm.at[pl.ds(0, d), pl.ds(0, CHUNK)], gbuf[b], gsem[b]
            ).wait()

        def wait_store(b):
            pltpu.make_async_copy(
                tbuf[b], out_hbm.at[pl.ds(0, 64)], ssem[b]
            ).wait()

        for b in range(ABUF - 1):
            fire(b, b)

        @pl.loop(0, (nfull + NW * ABUF - 1) // (NW * ABUF))
        def _(g):
            for b in range(ABUF):
                t = g * ABUF + b

                @pl.when(t < nblk)
                def _():
                    wait_gather(b)

                    @pl.when(t >= ABUF)
                    def _():
                        wait_store(b)

                    @plsc.parallel_loop(0, 16)
                    def _(o):
                        perm = jnp.bitwise_and(row16 + o, 15)
                        for j in range(d // 16):
                            for m in range(8):
                                vl = row16 + 16 * m
                                e = perm + 16 * j
                                val = plsc.load_gather(gbuf[b], [e, vl])
                                plsc.store_scatter(
                                    tbuf[b],
                                    [
                                        jax.lax.shift_right_logical(vl, 1),
                                        jnp.bitwise_and(vl, 1) * 64 + e,
                                    ],
                                    val,
                                )

                    pltpu.async_copy(
                        tbuf[b],
                        out_hbm.at[pl.ds((wid + t * NW) * 64, 64)],
                        ssem[b],
                    )
                    nxt = t + (ABUF - 1)

                    @pl.when(nxt < nblk)
                    def _():
                        fire((b + ABUF - 1) % ABUF, nxt)

        for b in range(ABUF):
            @pl.when(b < nblk)
            def _():
                wait_store(b)

    return ka(wt, wtail)


def _gather(x_flat, weight, n_tok, seq, d):
    b_per_w = n_tok * seq // NW
    nblock = b_per_w // CHUNK
    ngroup = nblock // NBUF
    blk_per_s = n_tok // CHUNK
    dt = d // 8
    mesh = plsc.VectorSubcoreMesh(core_axis_name="c", subcore_axis_name="s")

    @functools.partial(
        pl.kernel,
        out_type=jax.ShapeDtypeStruct((seq * dt * blk_per_s, 8, CHUNK), jnp.float32),
        mesh=mesh,
        scratch_types=(
            [pltpu.VMEM((b_per_w,), jnp.int32)]
            + [pltpu.VMEM((CHUNK, d), jnp.float32) for _ in range(NBUF)]
            + [pltpu.VMEM((d, CHUNK + 1), jnp.float32) for _ in range(NBUF)]
            + [pltpu.SemaphoreType.DMA for _ in range(2 * NBUF)]
        ),
        compiler_params=pltpu.CompilerParams(
            use_tc_tiling_on_sc=False, needs_layout_passes=False
        ),
    )
    def k(x_hbm, w_hbm, out_hbm, idx_v, *scratch):
        gbuf = scratch[:NBUF]
        tbuf = scratch[NBUF:2 * NBUF]
        gsem = scratch[2 * NBUF:3 * NBUF]
        ssem = scratch[3 * NBUF:]
        wid = lax.axis_index("s") * NC + lax.axis_index("c")
        base_blk = wid * nblock
        pltpu.sync_copy(x_hbm.at[pl.ds(base_blk * CHUNK, b_per_w)], idx_v)

        def fire(b, i):
            pltpu.async_copy(
                w_hbm.at[idx_v.at[pl.ds(i * CHUNK, CHUNK)]], gbuf[b], gsem[b]
            )

        def wait_gather(b):
            pltpu.make_async_copy(
                w_hbm.at[pl.ds(0, CHUNK)], gbuf[b], gsem[b]
            ).wait()

        def wait_stores(b):
            for jt in range(dt):
                pltpu.make_async_copy(
                    tbuf[b].at[pl.ds(jt * 8, 8), pl.ds(0, CHUNK)],
                    out_hbm.at[0],
                    ssem[b],
                ).wait()

        for b in range(LEAD):
            fire(b, b)

        row16 = lax.iota(jnp.int32, 16)

        @pl.loop(0, ngroup)
        def _(g):
            for b in range(NBUF):
                i = g * NBUF + b
                gblk = base_blk + i
                s = gblk // blk_per_s
                bt = gblk % blk_per_s
                wait_gather(b)

                @pl.when(i >= NBUF)
                def _():
                    wait_stores(b)

                @plsc.parallel_loop(0, CHUNK, unroll=8)
                def _(bq):
                    colv = row16 * 0 + bq
                    for j in range(d // 16):
                        v = gbuf[b][bq, pl.ds(16 * j, 16)]
                        plsc.store_scatter(
                            tbuf[b], [row16 + (16 * j), colv], v
                        )

                for jt in range(dt):
                    pltpu.async_copy(
                        tbuf[b].at[pl.ds(jt * 8, 8), pl.ds(0, CHUNK)],
                        out_hbm.at[(s * dt + jt) * blk_per_s + bt],
                        ssem[b],
                    )

                nxt = i + LEAD

                @pl.when(nxt < nblock)
                def _():
                    fire((b + LEAD) % NBUF, nxt)

        for b in range(NBUF):
            wait_stores(b)

    return k(x_flat, weight)


@functools.partial(jax.jit, static_argnums=(2, 3, 4))
def _pipeline(x_flat, weight, n_tok, seq, d):
    v = weight.shape[0]
    nfull = v // CHUNK
    wtail = weight[nfull * CHUNK:].reshape((v - nfull * CHUNK) * d // 128, 128)
    wsc = _relayout(weight.T, wtail).reshape(v, d)
    return _gather(x_flat, wsc, n_tok, seq, d)


def kernel(x, weight):
    n, seq = x.shape
    d = weight.shape[1]
    x_flat = x.T.reshape(n * seq).astype(jnp.int32)
    out = _pipeline(x_flat, weight, n, seq, d)
    return (
        out.reshape(seq, d // 8, n // CHUNK, 8, CHUNK)
        .transpose(2, 4, 0, 1, 3)
        .reshape(n, seq, d)
    )


# speedup vs baseline: 3.1647x; 1.0022x over previous
import functools

import jax
import jax.numpy as jnp
from jax import lax
from jax.experimental import pallas as pl
from jax.experimental.pallas import tpu as pltpu
from jax.experimental.pallas import tpu_sc as plsc

NC = 2
NS = 16
NW = NC * NS

CHUNK = 128
NBUF = 4
LEAD = 3
ABUF = 3


def _relayout(wt, wtail):
    d, v = wt.shape
    nfull = v // CHUNK
    mesh = plsc.VectorSubcoreMesh(core_axis_name="c", subcore_axis_name="s")

    @functools.partial(
        pl.kernel,
        out_type=jax.ShapeDtypeStruct((v // 2, 128), jnp.float32),
        mesh=mesh,
        scratch_types=(
            [pltpu.VMEM((d, 128), jnp.float32) for _ in range(ABUF)]
            + [pltpu.VMEM((64, 128), jnp.float32) for _ in range(ABUF)]
            + [pltpu.SemaphoreType.DMA for _ in range(2 * ABUF + 1)]
        ),
        compiler_params=pltpu.CompilerParams(
            use_tc_tiling_on_sc=True, needs_layout_passes=False
        ),
    )
    def ka(wt_hbm, wtail_hbm, out_hbm, *scratch):
        gbuf = scratch[:ABUF]
        tbuf = scratch[ABUF:2 * ABUF]
        gsem = scratch[2 * ABUF:3 * ABUF]
        ssem = scratch[3 * ABUF:4 * ABUF]
        tsem = scratch[4 * ABUF]
        wid = lax.axis_index("s") * NC + lax.axis_index("c")
        nblk = (nfull - wid + NW - 1) // NW
        row16 = lax.iota(jnp.int32, 16)

        @pl.when(wid == NW - 1)
        def _():
            pltpu.async_copy(wtail_hbm, gbuf[0].at[pl.ds(0, 32)], tsem)
            pltpu.make_async_copy(
                wtail_hbm, gbuf[0].at[pl.ds(0, 32)], tsem
            ).wait()
            pltpu.async_copy(
                gbuf[0].at[pl.ds(0, 32)],
                out_hbm.at[pl.ds(nfull * 64, 32)],
                tsem,
            )
            pltpu.make_async_copy(
                gbuf[0].at[pl.ds(0, 32)],
                out_hbm.at[pl.ds(nfull * 64, 32)],
                tsem,
            ).wait()

        def fire(b, t):
            pltpu.async_copy(
                wt_hbm.at[pl.ds(0, d), pl.ds((wid + t * NW) * CHUNK, CHUNK)],
                gbuf[b], gsem[b],
            )

        def wait_gather(b):
            pltpu.make_async_copy(
                wt_hbm.at[pl.ds(0, d), pl.ds(0, CHUNK)], gbuf[b], gsem[b]
            ).wait()

        def wait_store(b):
            pltpu.make_async_copy(
                tbuf[b], out_hbm.at[pl.ds(0, 64)], ssem[b]
            ).wait()

        for b in range(ABUF - 1):
            fire(b, b)

        @pl.loop(0, (nfull + NW * ABUF - 1) // (NW * ABUF))
        def _(g):
            for b in range(ABUF):
                t = g * ABUF + b

                @pl.when(t < nblk)
                def _():
                    wait_gather(b)

                    @pl.when(t >= ABUF)
                    def _():
                        wait_store(b)

                    vls = [row16 + 16 * m for m in range(8)]
                    rows = [
                        jax.lax.shift_right_logical(row16, 1) + 8 * m
                        for m in range(8)
                    ]
                    vlp = jnp.bitwise_and(row16, 1) * 64

                    @plsc.parallel_loop(0, 16)
                    def _(o):
                        perm = jnp.bitwise_and(row16 + o, 15)
                        colp = vlp + perm
                        for j in range(d // 16):
                            e = perm + 16 * j
                            col = colp + 16 * j
                            for m in range(8):
                                val = plsc.load_gather(gbuf[b], [e, vls[m]])
                                plsc.store_scatter(
                                    tbuf[b], [rows[m], col], val
                                )

                    pltpu.async_copy(
                        tbuf[b],
                        out_hbm.at[pl.ds((wid + t * NW) * 64, 64)],
                        ssem[b],
                    )
                    nxt = t + (ABUF - 1)

                    @pl.when(nxt < nblk)
                    def _():
                        fire((b + ABUF - 1) % ABUF, nxt)

        for b in range(ABUF):
            @pl.when(b < nblk)
            def _():
                wait_store(b)

    return ka(wt, wtail)


def _gather(x_flat, weight, n_tok, seq, d):
    b_per_w = n_tok * seq // NW
    nblock = b_per_w // CHUNK
    ngroup = nblock // NBUF
    blk_per_s = n_tok // CHUNK
    dt = d // 8
    mesh = plsc.VectorSubcoreMesh(core_axis_name="c", subcore_axis_name="s")

    @functools.partial(
        pl.kernel,
        out_type=jax.ShapeDtypeStruct((seq * dt * blk_per_s, 8, CHUNK), jnp.float32),
        mesh=mesh,
        scratch_types=(
            [pltpu.VMEM((b_per_w,), jnp.int32)]
            + [pltpu.VMEM((CHUNK, d), jnp.float32) for _ in range(NBUF)]
            + [pltpu.VMEM((d, CHUNK + 1), jnp.float32) for _ in range(NBUF)]
            + [pltpu.SemaphoreType.DMA for _ in range(2 * NBUF)]
        ),
        compiler_params=pltpu.CompilerParams(
            use_tc_tiling_on_sc=False, needs_layout_passes=False
        ),
    )
    def k(x_hbm, w_hbm, out_hbm, idx_v, *scratch):
        gbuf = scratch[:NBUF]
        tbuf = scratch[NBUF:2 * NBUF]
        gsem = scratch[2 * NBUF:3 * NBUF]
        ssem = scratch[3 * NBUF:]
        wid = lax.axis_index("s") * NC + lax.axis_index("c")
        base_blk = wid * nblock
        pltpu.sync_copy(x_hbm.at[pl.ds(base_blk * CHUNK, b_per_w)], idx_v)

        def fire(b, i):
            pltpu.async_copy(
                w_hbm.at[idx_v.at[pl.ds(i * CHUNK, CHUNK)]], gbuf[b], gsem[b]
            )

        def wait_gather(b):
            pltpu.make_async_copy(
                w_hbm.at[pl.ds(0, CHUNK)], gbuf[b], gsem[b]
            ).wait()

        def wait_stores(b):
            for jt in range(dt):
                pltpu.make_async_copy(
                    tbuf[b].at[pl.ds(jt * 8, 8), pl.ds(0, CHUNK)],
                    out_hbm.at[0],
                    ssem[b],
                ).wait()

        for b in range(LEAD):
            fire(b, b)

        row16 = lax.iota(jnp.int32, 16)

        @pl.loop(0, ngroup)
        def _(g):
            for b in range(NBUF):
                i = g * NBUF + b
                gblk = base_blk + i
                s = gblk // blk_per_s
                bt = gblk % blk_per_s
                wait_gather(b)

                @pl.when(i >= NBUF)
                def _():
                    wait_stores(b)

                @plsc.parallel_loop(0, CHUNK, unroll=8)
                def _(bq):
                    colv = row16 * 0 + bq
                    for j in range(d // 16):
                        v = gbuf[b][bq, pl.ds(16 * j, 16)]
                        plsc.store_scatter(
                            tbuf[b], [row16 + (16 * j), colv], v
                        )

                for jt in range(dt):
                    pltpu.async_copy(
                        tbuf[b].at[pl.ds(jt * 8, 8), pl.ds(0, CHUNK)],
                        out_hbm.at[(s * dt + jt) * blk_per_s + bt],
                        ssem[b],
                    )

                nxt = i + LEAD

                @pl.when(nxt < nblock)
                def _():
                    fire((b + LEAD) % NBUF, nxt)

        for b in range(NBUF):
            wait_stores(b)

    return k(x_flat, weight)


@functools.partial(jax.jit, static_argnums=(2, 3, 4))
def _pipeline(x_flat, weight, n_tok, seq, d):
    v = weight.shape[0]
    nfull = v // CHUNK
    wtail = weight[nfull * CHUNK:].reshape((v - nfull * CHUNK) * d // 128, 128)
    wsc = _relayout(weight.T, wtail).reshape(v, d)
    return _gather(x_flat, wsc, n_tok, seq, d)


def kernel(x, weight):
    n, seq = x.shape
    d = weight.shape[1]
    x_flat = x.T.reshape(n * seq).astype(jnp.int32)
    out = _pipeline(x_flat, weight, n, seq, d)
    return (
        out.reshape(seq, d // 8, n // CHUNK, 8, CHUNK)
        .transpose(2, 4, 0, 1, 3)
        .reshape(n, seq, d)
    )


# speedup vs baseline: 3.5015x; 1.1064x over previous
import functools

import jax
import jax.numpy as jnp
from jax import lax
from jax.experimental import pallas as pl
from jax.experimental.pallas import tpu as pltpu
from jax.experimental.pallas import tpu_sc as plsc

NC = 2
NS = 16
NW = NC * NS

CHUNK = 128
NBUF = 4
LEAD = 3
ABUF = 4


def _relayout(wt, wtail):
    d, v = wt.shape
    nfull = v // CHUNK
    mesh = plsc.VectorSubcoreMesh(core_axis_name="c", subcore_axis_name="s")

    @functools.partial(
        pl.kernel,
        out_type=jax.ShapeDtypeStruct((v // 2, 128), jnp.float32),
        mesh=mesh,
        scratch_types=(
            [pltpu.VMEM((d, 128), jnp.float32) for _ in range(ABUF)]
            + [pltpu.VMEM((64, 128), jnp.float32) for _ in range(ABUF)]
            + [pltpu.SemaphoreType.DMA for _ in range(2 * ABUF + 1)]
        ),
        compiler_params=pltpu.CompilerParams(
            use_tc_tiling_on_sc=True, needs_layout_passes=False
        ),
    )
    def ka(wt_hbm, wtail_hbm, out_hbm, *scratch):
        gbuf = scratch[:ABUF]
        tbuf = scratch[ABUF:2 * ABUF]
        gsem = scratch[2 * ABUF:3 * ABUF]
        ssem = scratch[3 * ABUF:4 * ABUF]
        tsem = scratch[4 * ABUF]
        wid = lax.axis_index("s") * NC + lax.axis_index("c")
        nblk = (nfull - wid + NW - 1) // NW
        row16 = lax.iota(jnp.int32, 16)

        @pl.when(wid == NW - 1)
        def _():
            pltpu.async_copy(wtail_hbm, gbuf[0].at[pl.ds(0, 32)], tsem)
            pltpu.make_async_copy(
                wtail_hbm, gbuf[0].at[pl.ds(0, 32)], tsem
            ).wait()
            pltpu.async_copy(
                gbuf[0].at[pl.ds(0, 32)],
                out_hbm.at[pl.ds(nfull * 64, 32)],
                tsem,
            )
            pltpu.make_async_copy(
                gbuf[0].at[pl.ds(0, 32)],
                out_hbm.at[pl.ds(nfull * 64, 32)],
                tsem,
            ).wait()

        def fire(b, t):
            pltpu.async_copy(
                wt_hbm.at[pl.ds(0, d), pl.ds((wid + t * NW) * CHUNK, CHUNK)],
                gbuf[b], gsem[b],
            )

        def wait_gather(b):
            pltpu.make_async_copy(
                wt_hbm.at[pl.ds(0, d), pl.ds(0, CHUNK)], gbuf[b], gsem[b]
            ).wait()

        def wait_store(b):
            pltpu.make_async_copy(
                tbuf[b], out_hbm.at[pl.ds(0, 64)], ssem[b]
            ).wait()

        for b in range(ABUF - 1):
            fire(b, b)

        @pl.loop(0, (nfull + NW * ABUF - 1) // (NW * ABUF))
        def _(g):
            for b in range(ABUF):
                t = g * ABUF + b

                @pl.when(t < nblk)
                def _():
                    wait_gather(b)

                    @pl.when(t >= ABUF)
                    def _():
                        wait_store(b)

                    vls = [row16 + 16 * m for m in range(8)]
                    rows = [
                        jax.lax.shift_right_logical(row16, 1) + 8 * m
                        for m in range(8)
                    ]
                    vlp = jnp.bitwise_and(row16, 1) * 64

                    @plsc.parallel_loop(0, 16, unroll=2)
                    def _(o):
                        perm = jnp.bitwise_and(row16 + o, 15)
                        colp = vlp + perm
                        for j in range(d // 16):
                            e = perm + 16 * j
                            col = colp + 16 * j
                            for m in range(8):
                                val = plsc.load_gather(gbuf[b], [e, vls[m]])
                                plsc.store_scatter(
                                    tbuf[b], [rows[m], col], val
                                )

                    pltpu.async_copy(
                        tbuf[b],
                        out_hbm.at[pl.ds((wid + t * NW) * 64, 64)],
                        ssem[b],
                    )
                    nxt = t + (ABUF - 1)

                    @pl.when(nxt < nblk)
                    def _():
                        fire((b + ABUF - 1) % ABUF, nxt)

        for b in range(ABUF):
            @pl.when(b < nblk)
            def _():
                wait_store(b)

    return ka(wt, wtail)


def _gather(x_flat, weight, n_tok, seq, d):
    b_per_w = n_tok * seq // NW
    nblock = b_per_w // CHUNK
    ngroup = nblock // NBUF
    blk_per_s = n_tok // CHUNK
    dt = d // 8
    mesh = plsc.VectorSubcoreMesh(core_axis_name="c", subcore_axis_name="s")

    @functools.partial(
        pl.kernel,
        out_type=jax.ShapeDtypeStruct((seq * dt * blk_per_s, 8, CHUNK), jnp.float32),
        mesh=mesh,
        scratch_types=(
            [pltpu.VMEM((b_per_w,), jnp.int32)]
            + [pltpu.VMEM((CHUNK, d), jnp.float32) for _ in range(NBUF)]
            + [pltpu.VMEM((d, CHUNK + 1), jnp.float32) for _ in range(NBUF)]
            + [pltpu.SemaphoreType.DMA for _ in range(2 * NBUF)]
        ),
        compiler_params=pltpu.CompilerParams(
            use_tc_tiling_on_sc=False, needs_layout_passes=False
        ),
    )
    def k(x_hbm, w_hbm, out_hbm, idx_v, *scratch):
        gbuf = scratch[:NBUF]
        tbuf = scratch[NBUF:2 * NBUF]
        gsem = scratch[2 * NBUF:3 * NBUF]
        ssem = scratch[3 * NBUF:]
        wid = lax.axis_index("s") * NC + lax.axis_index("c")
        base_blk = wid * nblock
        pltpu.sync_copy(x_hbm.at[pl.ds(base_blk * CHUNK, b_per_w)], idx_v)

        def fire(b, i):
            pltpu.async_copy(
                w_hbm.at[idx_v.at[pl.ds(i * CHUNK, CHUNK)]], gbuf[b], gsem[b]
            )

        def wait_gather(b):
            pltpu.make_async_copy(
                w_hbm.at[pl.ds(0, CHUNK)], gbuf[b], gsem[b]
            ).wait()

        def wait_stores(b):
            for jt in range(dt):
                pltpu.make_async_copy(
                    tbuf[b].at[pl.ds(jt * 8, 8), pl.ds(0, CHUNK)],
                    out_hbm.at[0],
                    ssem[b],
                ).wait()

        for b in range(LEAD):
            fire(b, b)

        row16 = lax.iota(jnp.int32, 16)

        @pl.loop(0, ngroup)
        def _(g):
            for b in range(NBUF):
                i = g * NBUF + b
                gblk = base_blk + i
                s = gblk // blk_per_s
                bt = gblk % blk_per_s
                wait_gather(b)

                @pl.when(i >= NBUF)
                def _():
                    wait_stores(b)

                @plsc.parallel_loop(0, CHUNK, unroll=8)
                def _(bq):
                    colv = row16 * 0 + bq
                    for j in range(d // 16):
                        v = gbuf[b][bq, pl.ds(16 * j, 16)]
                        plsc.store_scatter(
                            tbuf[b], [row16 + (16 * j), colv], v
                        )

                for jt in range(dt):
                    pltpu.async_copy(
                        tbuf[b].at[pl.ds(jt * 8, 8), pl.ds(0, CHUNK)],
                        out_hbm.at[(s * dt + jt) * blk_per_s + bt],
                        ssem[b],
                    )

                nxt = i + LEAD

                @pl.when(nxt < nblock)
                def _():
                    fire((b + LEAD) % NBUF, nxt)

        for b in range(NBUF):
            wait_stores(b)

    return k(x_flat, weight)


@functools.partial(jax.jit, static_argnums=(2, 3, 4))
def _pipeline(x_flat, weight, n_tok, seq, d):
    v = weight.shape[0]
    nfull = v // CHUNK
    wtail = weight[nfull * CHUNK:].reshape((v - nfull * CHUNK) * d // 128, 128)
    wsc = _relayout(weight.T, wtail).reshape(v, d)
    return _gather(x_flat, wsc, n_tok, seq, d)


def kernel(x, weight):
    n, seq = x.shape
    d = weight.shape[1]
    x_flat = x.T.reshape(n * seq).astype(jnp.int32)
    out = _pipeline(x_flat, weight, n, seq, d)
    return (
        out.reshape(seq, d // 8, n // CHUNK, 8, CHUNK)
        .transpose(2, 4, 0, 1, 3)
        .reshape(n, seq, d)
    )
